# Initial kernel scaffold; baseline (speedup 1.0000x reference)
import functools

import jax
import jax.numpy as jnp
from jax import lax
from jax.experimental import pallas as pl
from jax.experimental.pallas import tpu as pltpu
from jax.experimental.pallas import tpu_sc as plsc

GROUP = 128


@functools.partial(jax.jit, static_argnames=("n_total", "units"))
def _sc_gather(table, idx_flat, n_total, units):
    info = plsc.get_sparse_core_info()
    nw = info.num_cores * info.num_subcores
    n_groups = n_total // GROUP
    g_per_w = n_groups // nw
    mesh = plsc.VectorSubcoreMesh(core_axis_name="c", subcore_axis_name="s")

    @functools.partial(
        pl.kernel,
        out_type=jax.ShapeDtypeStruct((n_total, units), jnp.float32),
        mesh=mesh,
        scratch_types=[
            pltpu.VMEM((g_per_w, GROUP), jnp.int32),
            pltpu.VMEM((GROUP, units), jnp.float32),
            pltpu.SemaphoreType.DMA,
        ],
    )
    def k(table_hbm, idx_hbm, out_hbm, idx_v, rows_v, gsem):
        wid = lax.axis_index("s") * info.num_cores + lax.axis_index("c")
        gbase = wid * g_per_w
        pltpu.sync_copy(idx_hbm.at[pl.ds(gbase, g_per_w)], idx_v)

        @pl.loop(0, g_per_w)
        def _(j):
            pltpu.async_copy(table_hbm.at[idx_v.at[j]], rows_v, gsem).wait()
            pltpu.sync_copy(
                rows_v, out_hbm.at[pl.ds((gbase + j) * GROUP, GROUP)]
            )

    return k(table, idx_flat.reshape(n_groups, GROUP))


def kernel(inputs, kernel):
    batch, fields = inputs.shape
    n_total = batch * fields
    units = kernel.shape[1]
    out = _sc_gather(kernel, inputs.reshape(-1), n_total, units)
    return out.reshape(batch, fields, units)


# speedup vs baseline: 1.4376x; 1.4376x over previous
import functools

import jax
import jax.numpy as jnp
from jax import lax
from jax.experimental import pallas as pl
from jax.experimental.pallas import tpu as pltpu
from jax.experimental.pallas import tpu_sc as plsc

GROUP = 128


@functools.partial(jax.jit, static_argnames=("n_total", "units"))
def _sc_gather(table, idx_flat, n_total, units):
    info = plsc.get_sparse_core_info()
    nw = info.num_cores * info.num_subcores
    n_groups = n_total // GROUP
    g_per_w = n_groups // nw
    mesh = plsc.VectorSubcoreMesh(core_axis_name="c", subcore_axis_name="s")

    @functools.partial(
        pl.kernel,
        out_type=jax.ShapeDtypeStruct((n_total, units), jnp.float32),
        mesh=mesh,
        scratch_types=[
            pltpu.VMEM((g_per_w, GROUP), jnp.int32),
            pltpu.VMEM((GROUP, units), jnp.float32),
            pltpu.SemaphoreType.DMA,
        ],
        compiler_params=pltpu.CompilerParams(use_tc_tiling_on_sc=False),
    )
    def k(table_hbm, idx_hbm, out_hbm, idx_v, rows_v, gsem):
        wid = lax.axis_index("s") * info.num_cores + lax.axis_index("c")
        gbase = wid * g_per_w
        pltpu.sync_copy(idx_hbm.at[pl.ds(gbase, g_per_w)], idx_v)

        @pl.loop(0, g_per_w)
        def _(j):
            pltpu.async_copy(table_hbm.at[idx_v.at[j]], rows_v, gsem).wait()
            pltpu.sync_copy(
                rows_v, out_hbm.at[pl.ds((gbase + j) * GROUP, GROUP)]
            )

    return k(table, idx_flat.reshape(n_groups, GROUP))


def kernel(inputs, kernel):
    batch, fields = inputs.shape
    n_total = batch * fields
    units = kernel.shape[1]
    out = _sc_gather(kernel, inputs.reshape(-1), n_total, units)
    return out.reshape(batch, fields, units)


# speedup vs baseline: 1.5775x; 1.0973x over previous
import functools

import jax
import jax.numpy as jnp
from jax import lax
from jax.experimental import pallas as pl
from jax.experimental.pallas import tpu as pltpu
from jax.experimental.pallas import tpu_sc as plsc

GROUP = 128
STREAMS = 4
CHUNK = GROUP * STREAMS
NBUF = 2


@functools.partial(jax.jit, static_argnames=("n_total", "units"))
def _sc_gather(table, idx_flat, n_total, units):
    info = plsc.get_sparse_core_info()
    nw = info.num_cores * info.num_subcores
    n_groups = n_total // GROUP
    g_per_w = n_groups // nw
    n_chunks = g_per_w // STREAMS
    mesh = plsc.VectorSubcoreMesh(core_axis_name="c", subcore_axis_name="s")

    @functools.partial(
        pl.kernel,
        out_type=jax.ShapeDtypeStruct((n_total, units), jnp.float32),
        mesh=mesh,
        scratch_types=[
            pltpu.VMEM((g_per_w, GROUP), jnp.int32),
            pltpu.VMEM((NBUF, CHUNK, units), jnp.float32),
            [pltpu.SemaphoreType.DMA] * NBUF,
        ],
        compiler_params=pltpu.CompilerParams(use_tc_tiling_on_sc=False),
    )
    def k(table_hbm, idx_hbm, out_hbm, idx_v, rows_v, gsems):
        wid = lax.axis_index("s") * info.num_cores + lax.axis_index("c")
        gbase = wid * g_per_w
        pltpu.sync_copy(idx_hbm.at[pl.ds(gbase, g_per_w)], idx_v)

        def fire(c, b):
            for s in range(STREAMS):
                pltpu.async_copy(
                    table_hbm.at[idx_v.at[c * STREAMS + s]],
                    rows_v.at[b].at[pl.ds(s * GROUP, GROUP)],
                    gsems[b],
                )

        def drain(c, b):
            for s in range(STREAMS):
                pltpu.make_async_copy(
                    table_hbm.at[idx_v.at[c * STREAMS + s]],
                    rows_v.at[b].at[pl.ds(s * GROUP, GROUP)],
                    gsems[b],
                ).wait()
            pltpu.sync_copy(
                rows_v.at[b],
                out_hbm.at[pl.ds((gbase + c * STREAMS) * GROUP, CHUNK)],
            )

        for b in range(NBUF):
            fire(b, b)

        @pl.loop(0, n_chunks - NBUF, step=NBUF)
        def _(c0):
            for b in range(NBUF):
                drain(c0 + b, b)
                fire(c0 + b + NBUF, b)

        for b in range(NBUF):
            drain(n_chunks - NBUF + b, b)

    return k(table, idx_flat.reshape(n_groups, GROUP))


def kernel(inputs, kernel):
    batch, fields = inputs.shape
    n_total = batch * fields
    units = kernel.shape[1]
    out = _sc_gather(kernel, inputs.reshape(-1), n_total, units)
    return out.reshape(batch, fields, units)
